# baseline (speedup 1.0000x reference)
import functools

import jax
import jax.numpy as jnp
from jax.experimental import pallas as pl
from jax.experimental.pallas import tpu as pltpu
from jax.experimental.pallas import tpu_sc as plsc

EPS = 1e-5


def _fold(layers):
    out = []
    for (W, b, g, be) in layers:
        s = g / jnp.sqrt(1.0 + EPS)
        out.append(((W * s[:, None]).T, (b * s + be)[None, :]))
    return out


def _fps_body(ptsT_ref, out_ref, *, npoint):
    B = ptsT_ref.shape[0]
    N = ptsT_ref.shape[2]
    x = ptsT_ref[:, 0, :]
    y = ptsT_ref[:, 1, :]
    z = ptsT_ref[:, 2, :]
    iota = jax.lax.broadcasted_iota(jnp.int32, (B, N), 1)

    def body(i, carry):
        dist, far = carry
        sel = iota == far
        cx = jnp.sum(jnp.where(sel, x, 0.0), axis=1, keepdims=True)
        cy = jnp.sum(jnp.where(sel, y, 0.0), axis=1, keepdims=True)
        cz = jnp.sum(jnp.where(sel, z, 0.0), axis=1, keepdims=True)
        d = (x - cx) ** 2 + (y - cy) ** 2 + (z - cz) ** 2
        dist = jnp.minimum(dist, d)
        dmax = jnp.max(dist, axis=1, keepdims=True)
        far_new = jnp.min(jnp.where(dist == dmax, iota, N), axis=1,
                          keepdims=True)
        out_ref[pl.ds(i, 1), 0, :, :] = jnp.concatenate([cx, cy, cz],
                                                        axis=1)[None]
        return dist, far_new

    dist0 = jnp.full((B, N), 1e10, jnp.float32)
    far0 = jnp.zeros((B, 1), jnp.int32)
    jax.lax.fori_loop(0, npoint, body, (dist0, far0))


def _fps(ptsT, npoint):
    B, _, N = ptsT.shape
    out = pl.pallas_call(
        functools.partial(_fps_body, npoint=npoint),
        out_shape=jax.ShapeDtypeStruct((npoint, 1, B, 3), jnp.float32),
    )(ptsT)
    return jnp.transpose(out.reshape(npoint, B, 3), (1, 0, 2))


def _sa_body(pts_ref, ptsT_ref, feat_ref, nx_ref, w1_ref, b1_ref, w2_ref,
             b2_ref, w3_ref, b3_ref, out_ref, *, r2, K):
    pts = pts_ref[0]
    ptsT = ptsT_ref[0]
    feat = feat_ref[0]
    nx = nx_ref[0]
    N = pts.shape[0]
    S = nx.shape[0]
    w1 = w1_ref[...]
    b1 = b1_ref[...]

    Y = jnp.dot(jnp.concatenate([pts, feat], axis=1), w1,
                preferred_element_type=jnp.float32)
    Z = jnp.dot(nx, w1[0:3, :],
                preferred_element_type=jnp.float32)

    s_new = jnp.sum(nx * nx, axis=1, keepdims=True)
    s_src = jnp.sum(ptsT * ptsT, axis=0, keepdims=True)
    cross = jnp.dot(nx, ptsT, preferred_element_type=jnp.float32)
    sqd = (s_new + s_src) - 2.0 * cross

    mask = sqd <= r2
    r = mask.astype(jnp.int16)
    sh = 1
    while sh < N:
        r = r + jnp.concatenate(
            [jnp.zeros((S, sh), jnp.int16), r[:, : N - sh]], axis=1)
        sh *= 2
    cnt = r[:, N - 1: N].astype(jnp.int32)

    t = jnp.where(mask, r, jnp.int16(0))
    tb = t.astype(jnp.bfloat16)
    Yb = Y.astype(jnp.bfloat16)
    one_b = jnp.bfloat16(1.0)
    zero_b = jnp.bfloat16(0.0)
    hs = []
    for k in range(K):
        oh = jnp.where(tb == jnp.bfloat16(k + 1), one_b, zero_b)
        g = jnp.dot(oh, Yb, preferred_element_type=jnp.float32)
        hs.append(jnp.maximum(g - Z + b1, 0.0))
    H = jnp.concatenate(hs, axis=0)

    H = jnp.maximum(
        jnp.dot(H, w2_ref[...], preferred_element_type=jnp.float32)
        + b2_ref[...], 0.0)
    H = jnp.maximum(
        jnp.dot(H, w3_ref[...], preferred_element_type=jnp.float32)
        + b3_ref[...], 0.0)
    C3 = H.shape[1]
    H = H.reshape(K, S, C3)
    kio = jax.lax.broadcasted_iota(jnp.int32, (K, S, 1), 0)
    out_ref[0] = jnp.max(jnp.where(cnt[None, :, :] > kio, H, 0.0), axis=0)


def _sa(pts, feats, new_xyz, layers, radius, K):
    B, N, _ = pts.shape
    Cf = feats.shape[2]
    S = new_xyz.shape[1]
    (w1, b1), (w2, b2), (w3, b3) = layers
    C3 = w3.shape[1]
    ptsT = jnp.transpose(pts, (0, 2, 1))
    return pl.pallas_call(
        functools.partial(_sa_body, r2=radius ** 2, K=K),
        grid=(B,),
        in_specs=[
            pl.BlockSpec((1, N, 3), lambda b: (b, 0, 0)),
            pl.BlockSpec((1, 3, N), lambda b: (b, 0, 0)),
            pl.BlockSpec((1, N, Cf), lambda b: (b, 0, 0)),
            pl.BlockSpec((1, S, 3), lambda b: (b, 0, 0)),
            pl.BlockSpec(w1.shape, lambda b: (0, 0)),
            pl.BlockSpec(b1.shape, lambda b: (0, 0)),
            pl.BlockSpec(w2.shape, lambda b: (0, 0)),
            pl.BlockSpec(b2.shape, lambda b: (0, 0)),
            pl.BlockSpec(w3.shape, lambda b: (0, 0)),
            pl.BlockSpec(b3.shape, lambda b: (0, 0)),
        ],
        out_specs=pl.BlockSpec((1, S, C3), lambda b: (b, 0, 0)),
        out_shape=jax.ShapeDtypeStruct((B, S, C3), jnp.float32),
    )(pts, ptsT, feats, new_xyz, w1, b1, w2, b2, w3, b3)


def _sa_pre_body(pts_ref, ptsT_ref, feat_ref, nx_ref, w1_ref, y_ref, z_ref,
                 cnt_ref, idx_ref, *, r2, K):
    b = pl.program_id(0)
    pts = pts_ref[0]
    ptsT = ptsT_ref[0]
    feat = feat_ref[0]
    nx = nx_ref[0]
    N = pts.shape[0]
    S = nx.shape[0]
    w1 = w1_ref[...]

    Y = jnp.dot(jnp.concatenate([pts, feat], axis=1), w1,
                preferred_element_type=jnp.float32)
    C1 = Y.shape[1]
    if C1 < 128:
        Y = jnp.concatenate(
            [Y, jnp.zeros((N, 128 - C1), jnp.float32)], axis=1)
    y_ref[0] = Y
    z_ref[0] = jnp.dot(nx, w1[0:3, :],
                       preferred_element_type=jnp.float32)

    s_new = jnp.sum(nx * nx, axis=1, keepdims=True)
    s_src = jnp.sum(ptsT * ptsT, axis=0, keepdims=True)
    cross = jnp.dot(nx, ptsT, preferred_element_type=jnp.float32)
    sqd = (s_new + s_src) - 2.0 * cross

    mask = sqd <= r2
    r = mask.astype(jnp.int16)
    sh = 1
    while sh < N:
        r = r + jnp.concatenate(
            [jnp.zeros((S, sh), jnp.int16), r[:, : N - sh]], axis=1)
        sh *= 2
    cnt_ref[0] = r[:, N - 1: N].astype(jnp.int32)

    t = jnp.where(mask, r, jnp.int16(0)).astype(jnp.float32)
    iota_n = jax.lax.broadcasted_iota(
        jnp.int32, (S, N), 1).astype(jnp.float32)
    big = jnp.float32(N)
    ms = []
    for k in range(K):
        val = jnp.where(t == jnp.float32(k + 1), iota_n, big)
        ms.append(jnp.min(val, axis=1, keepdims=True))
    idxi = jnp.concatenate(ms, axis=1).astype(jnp.int32)
    idx_ref[0] = jnp.where(idxi >= N, 0, idxi) + b * N


def _sa_pre(pts, feats, new_xyz, w1, radius, K):
    B, N, _ = pts.shape
    Cf = feats.shape[2]
    S = new_xyz.shape[1]
    C1 = w1.shape[1]
    ptsT = jnp.transpose(pts, (0, 2, 1))
    return pl.pallas_call(
        functools.partial(_sa_pre_body, r2=radius ** 2, K=K),
        grid=(B,),
        in_specs=[
            pl.BlockSpec((1, N, 3), lambda b: (b, 0, 0)),
            pl.BlockSpec((1, 3, N), lambda b: (b, 0, 0)),
            pl.BlockSpec((1, N, Cf), lambda b: (b, 0, 0)),
            pl.BlockSpec((1, S, 3), lambda b: (b, 0, 0)),
            pl.BlockSpec(w1.shape, lambda b: (0, 0)),
        ],
        out_specs=[
            pl.BlockSpec((1, N, max(C1, 128)), lambda b: (b, 0, 0)),
            pl.BlockSpec((1, S, C1), lambda b: (b, 0, 0)),
            pl.BlockSpec((1, S, 1), lambda b: (b, 0, 0)),
            pl.BlockSpec((1, S, K), lambda b: (b, 0, 0)),
        ],
        out_shape=[
            jax.ShapeDtypeStruct((B, N, max(C1, 128)), jnp.float32),
            jax.ShapeDtypeStruct((B, S, C1), jnp.float32),
            jax.ShapeDtypeStruct((B, S, 1), jnp.int32),
            jax.ShapeDtypeStruct((B, S, K), jnp.int32),
        ],
    )(pts, ptsT, feats, new_xyz, w1)


def _sc_gather(table, idx):
    R, D = table.shape
    M = idx.shape[0]
    info = plsc.get_sparse_core_info()
    nc = info.num_cores
    nw = nc * info.num_subcores
    m_per_w = M // nw
    ch = min(512, m_per_w)
    n_it = m_per_w // ch
    mesh = plsc.VectorSubcoreMesh(core_axis_name="c", subcore_axis_name="s")

    @functools.partial(
        pl.kernel, mesh=mesh,
        out_type=jax.ShapeDtypeStruct((M, D), jnp.float32),
        scratch_types=[
            pltpu.VMEM((ch,), jnp.int32),
            pltpu.VMEM((ch, D), jnp.float32),
            pltpu.SemaphoreType.DMA,
        ],
    )
    def gk(table_hbm, idx_hbm, out_hbm, idx_v, rows_v, sem):
        wid = jax.lax.axis_index("s") * nc + jax.lax.axis_index("c")
        base = wid * m_per_w
        for c in range(n_it):
            off = base + c * ch
            pltpu.sync_copy(idx_hbm.at[pl.ds(off, ch)], idx_v)
            pltpu.async_copy(table_hbm.at[idx_v], rows_v, sem).wait()
            pltpu.sync_copy(rows_v, out_hbm.at[pl.ds(off, ch)])

    return gk(table, idx)


def _sa_mlp_body(g_ref, z_ref, cnt_ref, b1_ref, w2_ref, b2_ref, w3_ref,
                 b3_ref, out_ref, *, K):
    Z = z_ref[0]
    S, C1 = Z.shape
    G = g_ref[0][:, :C1]
    Zr = jnp.broadcast_to(Z[:, None, :], (S, K, C1)).reshape(S * K, C1)
    H = jnp.maximum(G - Zr + b1_ref[...], 0.0)
    H = jnp.maximum(
        jnp.dot(H, w2_ref[...], preferred_element_type=jnp.float32)
        + b2_ref[...], 0.0)
    H = jnp.maximum(
        jnp.dot(H, w3_ref[...], preferred_element_type=jnp.float32)
        + b3_ref[...], 0.0)
    C3 = H.shape[1]
    H = H.reshape(S, K, C3)
    kio = jax.lax.broadcasted_iota(jnp.int32, (S, K, 1), 1)
    valid = cnt_ref[0][:, :, None] > kio
    out_ref[0] = jnp.max(jnp.where(valid, H, 0.0), axis=1)


def _sa_mlp(G, Z, cnt, layers, K):
    B, SK, D = G.shape
    C1 = Z.shape[2]
    S = SK // K
    (_, b1), (w2, b2), (w3, b3) = layers
    C3 = w3.shape[1]
    return pl.pallas_call(
        functools.partial(_sa_mlp_body, K=K),
        grid=(B,),
        in_specs=[
            pl.BlockSpec((1, SK, D), lambda b: (b, 0, 0)),
            pl.BlockSpec((1, S, C1), lambda b: (b, 0, 0)),
            pl.BlockSpec((1, S, 1), lambda b: (b, 0, 0)),
            pl.BlockSpec(b1.shape, lambda b: (0, 0)),
            pl.BlockSpec(w2.shape, lambda b: (0, 0)),
            pl.BlockSpec(b2.shape, lambda b: (0, 0)),
            pl.BlockSpec(w3.shape, lambda b: (0, 0)),
            pl.BlockSpec(b3.shape, lambda b: (0, 0)),
        ],
        out_specs=pl.BlockSpec((1, S, C3), lambda b: (b, 0, 0)),
        out_shape=jax.ShapeDtypeStruct((B, S, C3), jnp.float32),
    )(G, Z, cnt, b1, w2, b2, w3, b3)


def _sa3_body(xyz_ref, feat_ref, w1_ref, b1_ref, w2_ref, b2_ref, w3_ref,
              b3_ref, out_ref, *, B, M):
    x = jnp.concatenate([xyz_ref[...], feat_ref[...]], axis=1)
    h = jnp.maximum(
        jnp.dot(x, w1_ref[...], preferred_element_type=jnp.float32)
        + b1_ref[...], 0.0)
    h = jnp.maximum(
        jnp.dot(h, w2_ref[...], preferred_element_type=jnp.float32)
        + b2_ref[...], 0.0)
    h = jnp.maximum(
        jnp.dot(h, w3_ref[...], preferred_element_type=jnp.float32)
        + b3_ref[...], 0.0)
    out_ref[...] = jnp.max(h.reshape(B, M, h.shape[1]), axis=1)


def _sa3(l_xyz, l_points, layers):
    B, M, _ = l_xyz.shape
    (w1, b1), (w2, b2), (w3, b3) = layers
    C3 = w3.shape[1]
    return pl.pallas_call(
        functools.partial(_sa3_body, B=B, M=M),
        out_shape=jax.ShapeDtypeStruct((B, C3), jnp.float32),
    )(l_xyz.reshape(B * M, 3), l_points.reshape(B * M, -1),
      w1, b1, w2, b2, w3, b3)


def kernel(xyz, params):
    B = xyz.shape[0]
    l0_xyz = jnp.transpose(xyz, (0, 2, 1))
    sa1 = _fold(params['sa1'])
    sa2 = _fold(params['sa2'])
    sa3 = _fold(params['sa3'])

    l1_xyz = _fps(xyz, 512)
    N, S1, K1 = 4096, 512, 32
    Y, Zc, cnt, idxg = _sa_pre(l0_xyz, l0_xyz, l1_xyz, sa1[0][0], 0.2, K1)
    D = Y.shape[2]
    G = _sc_gather(Y.reshape(B * N, D), idxg.reshape(B * S1 * K1))
    l1_points = _sa_mlp(G.reshape(B, S1 * K1, D), Zc, cnt, sa1, K1)
    l2_xyz = _fps(jnp.transpose(l1_xyz, (0, 2, 1)), 128)
    l2_points = _sa(l1_xyz, l1_points, l2_xyz, sa2, 0.4, 64)
    return _sa3(l2_xyz, l2_points, sa3)


# speedup vs baseline: 1.8964x; 1.8964x over previous
import functools

import jax
import jax.numpy as jnp
from jax.experimental import pallas as pl
from jax.experimental.pallas import tpu as pltpu

EPS = 1e-5


def _fold(layers):
    out = []
    for (W, b, g, be) in layers:
        s = g / jnp.sqrt(1.0 + EPS)
        out.append(((W * s[:, None]).T, (b * s + be)[None, :]))
    return out


def _fps_body(ptsT_ref, out_ref, *, npoint):
    B = ptsT_ref.shape[0]
    N = ptsT_ref.shape[2]
    x = ptsT_ref[:, 0, :]
    y = ptsT_ref[:, 1, :]
    z = ptsT_ref[:, 2, :]
    iota = jax.lax.broadcasted_iota(jnp.int32, (B, N), 1)

    def body(i, carry):
        dist, far = carry
        sel = iota == far
        cx = jnp.sum(jnp.where(sel, x, 0.0), axis=1, keepdims=True)
        cy = jnp.sum(jnp.where(sel, y, 0.0), axis=1, keepdims=True)
        cz = jnp.sum(jnp.where(sel, z, 0.0), axis=1, keepdims=True)
        d = (x - cx) ** 2 + (y - cy) ** 2 + (z - cz) ** 2
        dist = jnp.minimum(dist, d)
        far_new = jnp.argmax(dist, axis=1).astype(jnp.int32)[:, None]
        out_ref[pl.ds(i, 1), 0, :, :] = jnp.concatenate([cx, cy, cz],
                                                        axis=1)[None]
        return dist, far_new

    dist0 = jnp.full((B, N), 1e10, jnp.float32)
    far0 = jnp.zeros((B, 1), jnp.int32)
    jax.lax.fori_loop(0, npoint, body, (dist0, far0))


def _fps(ptsT, npoint):
    B, _, N = ptsT.shape
    out = pl.pallas_call(
        functools.partial(_fps_body, npoint=npoint),
        out_shape=jax.ShapeDtypeStruct((npoint, 1, B, 3), jnp.float32),
    )(ptsT)
    return jnp.transpose(out.reshape(npoint, B, 3), (1, 0, 2))


def _sa_body(pts_ref, ptsT_ref, feat_ref, nx_ref, w1_ref, b1_ref, w2_ref,
             b2_ref, w3_ref, b3_ref, out_ref, *, r2, K):
    pts = pts_ref[0]
    ptsT = ptsT_ref[0]
    feat = feat_ref[0]
    nx = nx_ref[0]
    N = pts.shape[0]
    S = nx.shape[0]
    w1 = w1_ref[...]
    b1 = b1_ref[...]

    Y = jnp.dot(jnp.concatenate([pts, feat], axis=1), w1,
                preferred_element_type=jnp.float32)
    Z = jnp.dot(nx, w1[0:3, :],
                preferred_element_type=jnp.float32)

    s_new = jnp.sum(nx * nx, axis=1, keepdims=True)
    s_src = jnp.sum(ptsT * ptsT, axis=0, keepdims=True)
    cross = jnp.dot(nx, ptsT, preferred_element_type=jnp.float32)
    sqd = (s_new + s_src) - 2.0 * cross

    mask = sqd <= r2
    r = mask.astype(jnp.int16)
    sh = 1
    while sh < N:
        r = r + jnp.concatenate(
            [jnp.zeros((S, sh), jnp.int16), r[:, : N - sh]], axis=1)
        sh *= 2
    cnt = r[:, N - 1: N].astype(jnp.int32)

    t = jnp.where(mask, r, jnp.int16(0))
    tb = t.astype(jnp.bfloat16)
    Yb = Y.astype(jnp.bfloat16)
    one_b = jnp.bfloat16(1.0)
    zero_b = jnp.bfloat16(0.0)
    hs = []
    for k in range(K):
        oh = jnp.where(tb == jnp.bfloat16(k + 1), one_b, zero_b)
        g = jnp.dot(oh, Yb, preferred_element_type=jnp.float32)
        hs.append(jnp.maximum(g - Z + b1, 0.0))
    H = jnp.concatenate(hs, axis=0)

    H = jnp.maximum(
        jnp.dot(H, w2_ref[...], preferred_element_type=jnp.float32)
        + b2_ref[...], 0.0)
    H = jnp.maximum(
        jnp.dot(H, w3_ref[...], preferred_element_type=jnp.float32)
        + b3_ref[...], 0.0)
    C3 = H.shape[1]
    H = H.reshape(K, S, C3)
    kio = jax.lax.broadcasted_iota(jnp.int32, (K, S, 1), 0)
    out_ref[0] = jnp.max(jnp.where(cnt[None, :, :] > kio, H, 0.0), axis=0)


def _sa(pts, feats, new_xyz, layers, radius, K):
    B, N, _ = pts.shape
    Cf = feats.shape[2]
    S = new_xyz.shape[1]
    (w1, b1), (w2, b2), (w3, b3) = layers
    C3 = w3.shape[1]
    ptsT = jnp.transpose(pts, (0, 2, 1))
    return pl.pallas_call(
        functools.partial(_sa_body, r2=radius ** 2, K=K),
        grid=(B,),
        in_specs=[
            pl.BlockSpec((1, N, 3), lambda b: (b, 0, 0)),
            pl.BlockSpec((1, 3, N), lambda b: (b, 0, 0)),
            pl.BlockSpec((1, N, Cf), lambda b: (b, 0, 0)),
            pl.BlockSpec((1, S, 3), lambda b: (b, 0, 0)),
            pl.BlockSpec(w1.shape, lambda b: (0, 0)),
            pl.BlockSpec(b1.shape, lambda b: (0, 0)),
            pl.BlockSpec(w2.shape, lambda b: (0, 0)),
            pl.BlockSpec(b2.shape, lambda b: (0, 0)),
            pl.BlockSpec(w3.shape, lambda b: (0, 0)),
            pl.BlockSpec(b3.shape, lambda b: (0, 0)),
        ],
        out_specs=pl.BlockSpec((1, S, C3), lambda b: (b, 0, 0)),
        out_shape=jax.ShapeDtypeStruct((B, S, C3), jnp.float32),
    )(pts, ptsT, feats, new_xyz, w1, b1, w2, b2, w3, b3)


def _sa3_body(xyz_ref, feat_ref, w1_ref, b1_ref, w2_ref, b2_ref, w3_ref,
              b3_ref, out_ref, *, B, M):
    x = jnp.concatenate([xyz_ref[...], feat_ref[...]], axis=1)
    h = jnp.maximum(
        jnp.dot(x, w1_ref[...], preferred_element_type=jnp.float32)
        + b1_ref[...], 0.0)
    h = jnp.maximum(
        jnp.dot(h, w2_ref[...], preferred_element_type=jnp.float32)
        + b2_ref[...], 0.0)
    h = jnp.maximum(
        jnp.dot(h, w3_ref[...], preferred_element_type=jnp.float32)
        + b3_ref[...], 0.0)
    out_ref[...] = jnp.max(h.reshape(B, M, h.shape[1]), axis=1)


def _sa3(l_xyz, l_points, layers):
    B, M, _ = l_xyz.shape
    (w1, b1), (w2, b2), (w3, b3) = layers
    C3 = w3.shape[1]
    return pl.pallas_call(
        functools.partial(_sa3_body, B=B, M=M),
        out_shape=jax.ShapeDtypeStruct((B, C3), jnp.float32),
    )(l_xyz.reshape(B * M, 3), l_points.reshape(B * M, -1),
      w1, b1, w2, b2, w3, b3)


def kernel(xyz, params):
    B = xyz.shape[0]
    l0_xyz = jnp.transpose(xyz, (0, 2, 1))
    sa1 = _fold(params['sa1'])
    sa2 = _fold(params['sa2'])
    sa3 = _fold(params['sa3'])

    l1_xyz = _fps(xyz, 512)
    l1_points = _sa(l0_xyz, l0_xyz, l1_xyz, sa1, 0.2, 32)
    l2_xyz = _fps(jnp.transpose(l1_xyz, (0, 2, 1)), 128)
    l2_points = _sa(l1_xyz, l1_points, l2_xyz, sa2, 0.4, 64)
    return _sa3(l2_xyz, l2_points, sa3)


# speedup vs baseline: 1.9854x; 1.0469x over previous
import functools

import jax
import jax.numpy as jnp
from jax.experimental import pallas as pl
from jax.experimental.pallas import tpu as pltpu

EPS = 1e-5


def _fold(layers):
    out = []
    for (W, b, g, be) in layers:
        s = g / jnp.sqrt(1.0 + EPS)
        out.append(((W * s[:, None]).T, (b * s + be)[None, :]))
    return out


def _fps_body(ptsT_ref, out_ref, *, npoint):
    B = ptsT_ref.shape[0]
    N = ptsT_ref.shape[2]
    x = ptsT_ref[:, 0, :]
    y = ptsT_ref[:, 1, :]
    z = ptsT_ref[:, 2, :]
    iota = jax.lax.broadcasted_iota(jnp.int32, (B, N), 1)

    def body(i, carry):
        dist, far = carry
        sel = iota == far
        cx = jnp.sum(jnp.where(sel, x, 0.0), axis=1, keepdims=True)
        cy = jnp.sum(jnp.where(sel, y, 0.0), axis=1, keepdims=True)
        cz = jnp.sum(jnp.where(sel, z, 0.0), axis=1, keepdims=True)
        d = (x - cx) ** 2 + (y - cy) ** 2 + (z - cz) ** 2
        dist = jnp.minimum(dist, d)
        far_new = jnp.argmax(dist, axis=1).astype(jnp.int32)[:, None]
        out_ref[pl.ds(i, 1), 0, :, :] = jnp.concatenate([cx, cy, cz],
                                                        axis=1)[None]
        return dist, far_new

    dist0 = jnp.full((B, N), 1e10, jnp.float32)
    far0 = jnp.zeros((B, 1), jnp.int32)
    jax.lax.fori_loop(0, npoint, body, (dist0, far0))


def _fps(ptsT, npoint):
    B, _, N = ptsT.shape
    out = pl.pallas_call(
        functools.partial(_fps_body, npoint=npoint),
        out_shape=jax.ShapeDtypeStruct((npoint, 1, B, 3), jnp.float32),
    )(ptsT)
    return jnp.transpose(out.reshape(npoint, B, 3), (1, 0, 2))


def _sa_body(pts_ref, ptsT_ref, feat_ref, nx_ref, w1_ref, b1_ref, w2_ref,
             b2_ref, w3_ref, b3_ref, out_ref, *, r2, K):
    pts = pts_ref[0]
    ptsT = ptsT_ref[0]
    feat = feat_ref[0]
    nx = nx_ref[0]
    N = pts.shape[0]
    S = nx.shape[0]
    w1 = w1_ref[...]
    b1 = b1_ref[...]

    Y = jnp.dot(jnp.concatenate([pts, feat], axis=1), w1,
                preferred_element_type=jnp.float32)
    Z = jnp.dot(nx, w1[0:3, :],
                preferred_element_type=jnp.float32)

    s_new = jnp.sum(nx * nx, axis=1, keepdims=True)
    s_src = jnp.sum(ptsT * ptsT, axis=0, keepdims=True)
    cross = jnp.dot(nx, ptsT, preferred_element_type=jnp.float32)
    sqd = (s_new + s_src) - 2.0 * cross

    mask = sqd <= r2
    r = mask.astype(jnp.int16)
    sh = 1
    while sh < N:
        r = r + jnp.concatenate(
            [jnp.zeros((S, sh), jnp.int16), r[:, : N - sh]], axis=1)
        sh *= 2
    cnt = r[:, N - 1: N].astype(jnp.int32)

    t = jnp.where(mask, r, jnp.int16(0))
    tb = t.astype(jnp.bfloat16)
    Yb = Y.astype(jnp.bfloat16)
    one_b = jnp.bfloat16(1.0)
    zero_b = jnp.bfloat16(0.0)
    GRP = 8
    Zg = jnp.concatenate([Z - b1] * GRP, axis=0)
    hs = []
    for k0 in range(0, K, GRP):
        oh = jnp.concatenate(
            [jnp.where(tb == jnp.bfloat16(k + 1), one_b, zero_b)
             for k in range(k0, k0 + GRP)], axis=0)
        g = jnp.dot(oh, Yb, preferred_element_type=jnp.float32)
        hs.append(jnp.maximum(g - Zg, 0.0))
    H = jnp.concatenate(hs, axis=0)

    H = jnp.maximum(
        jnp.dot(H.astype(jnp.bfloat16), w2_ref[...].astype(jnp.bfloat16),
                preferred_element_type=jnp.float32)
        + b2_ref[...], 0.0)
    H = jnp.maximum(
        jnp.dot(H.astype(jnp.bfloat16), w3_ref[...].astype(jnp.bfloat16),
                preferred_element_type=jnp.float32)
        + b3_ref[...], 0.0)
    C3 = H.shape[1]
    H = H.reshape(K, S, C3)
    kio = jax.lax.broadcasted_iota(jnp.int32, (K, S, 1), 0)
    out_ref[0] = jnp.max(jnp.where(cnt[None, :, :] > kio, H, 0.0), axis=0)


def _sa(pts, feats, new_xyz, layers, radius, K):
    B, N, _ = pts.shape
    Cf = feats.shape[2]
    S = new_xyz.shape[1]
    (w1, b1), (w2, b2), (w3, b3) = layers
    C3 = w3.shape[1]
    ptsT = jnp.transpose(pts, (0, 2, 1))
    return pl.pallas_call(
        functools.partial(_sa_body, r2=radius ** 2, K=K),
        grid=(B,),
        in_specs=[
            pl.BlockSpec((1, N, 3), lambda b: (b, 0, 0)),
            pl.BlockSpec((1, 3, N), lambda b: (b, 0, 0)),
            pl.BlockSpec((1, N, Cf), lambda b: (b, 0, 0)),
            pl.BlockSpec((1, S, 3), lambda b: (b, 0, 0)),
            pl.BlockSpec(w1.shape, lambda b: (0, 0)),
            pl.BlockSpec(b1.shape, lambda b: (0, 0)),
            pl.BlockSpec(w2.shape, lambda b: (0, 0)),
            pl.BlockSpec(b2.shape, lambda b: (0, 0)),
            pl.BlockSpec(w3.shape, lambda b: (0, 0)),
            pl.BlockSpec(b3.shape, lambda b: (0, 0)),
        ],
        out_specs=pl.BlockSpec((1, S, C3), lambda b: (b, 0, 0)),
        out_shape=jax.ShapeDtypeStruct((B, S, C3), jnp.float32),
    )(pts, ptsT, feats, new_xyz, w1, b1, w2, b2, w3, b3)


def _sa3_body(xyz_ref, feat_ref, w1_ref, b1_ref, w2_ref, b2_ref, w3_ref,
              b3_ref, out_ref, *, B, M):
    x = jnp.concatenate([xyz_ref[...], feat_ref[...]], axis=1)
    h = jnp.maximum(
        jnp.dot(x, w1_ref[...], preferred_element_type=jnp.float32)
        + b1_ref[...], 0.0)
    h = jnp.maximum(
        jnp.dot(h, w2_ref[...], preferred_element_type=jnp.float32)
        + b2_ref[...], 0.0)
    h = jnp.maximum(
        jnp.dot(h, w3_ref[...], preferred_element_type=jnp.float32)
        + b3_ref[...], 0.0)
    out_ref[...] = jnp.max(h.reshape(B, M, h.shape[1]), axis=1)


def _sa3(l_xyz, l_points, layers):
    B, M, _ = l_xyz.shape
    (w1, b1), (w2, b2), (w3, b3) = layers
    C3 = w3.shape[1]
    return pl.pallas_call(
        functools.partial(_sa3_body, B=B, M=M),
        out_shape=jax.ShapeDtypeStruct((B, C3), jnp.float32),
    )(l_xyz.reshape(B * M, 3), l_points.reshape(B * M, -1),
      w1, b1, w2, b2, w3, b3)


def kernel(xyz, params):
    B = xyz.shape[0]
    l0_xyz = jnp.transpose(xyz, (0, 2, 1))
    sa1 = _fold(params['sa1'])
    sa2 = _fold(params['sa2'])
    sa3 = _fold(params['sa3'])

    l1_xyz = _fps(xyz, 512)
    l1_points = _sa(l0_xyz, l0_xyz, l1_xyz, sa1, 0.2, 32)
    l2_xyz = _fps(jnp.transpose(l1_xyz, (0, 2, 1)), 128)
    l2_points = _sa(l1_xyz, l1_points, l2_xyz, sa2, 0.4, 64)
    return _sa3(l2_xyz, l2_points, sa3)
